# baseline (device time: 38536 ns/iter reference)
import jax
import jax.numpy as jnp
from jax import lax
from jax.experimental import pallas as pl
from jax.experimental.pallas import tpu as pltpu
import functools

N_DEV = 8
H_LOC = 8
DH = 128
SQ = 256
SKV = 4096
QB = 64
NQB = SQ // QB
STRIDE = 4
NKB = SKV // QB
KV_SEL = (NKB // STRIDE) * QB
CHUNK = SQ // N_DEV
DM = 1024
SCALE = 0.08838834764831843
BF = jnp.bfloat16


def kernel(x, Wq, K_ext, V_ext, Wo):
    def body(x_ref, wq_ref, k_ref, v_ref, wo_ref, out_ref,
             wq_v, wo_v, k_all, v_all, q_v, ctx_v, partA, partB, red_v,
             p1bufA, p1bufB, p2buf,
             w_sems, k_sems, v_sems, p1A_send, p1A_recv, p1B_send,
             p1B_recv, p2_send, p2_recv):
        my = lax.axis_index("i")
        DORD = [4, 3, 5, 2, 6, 1, 7]

        wq_cp = pltpu.make_async_copy(
            wq_ref.at[:, pl.ds(my * DM, DM)], wq_v, w_sems.at[0])
        wo_cp = pltpu.make_async_copy(
            wo_ref.at[pl.ds(my * DM, DM), :], wo_v, w_sems.at[1])
        wq_cp.start()
        wo_cp.start()

        head_cps = []
        for h in range(H_LOC):
            k_cp = pltpu.make_async_copy(k_ref.at[0, :, h, :],
                                         k_all.at[h], k_sems.at[h])
            v_cp = pltpu.make_async_copy(v_ref.at[0, :, h, :],
                                         v_all.at[h], v_sems.at[h])
            k_cp.start()
            v_cp.start()
            head_cps.append((k_cp, v_cp))

        bsem = pltpu.get_barrier_semaphore()
        for d in range(1, N_DEV):
            pl.semaphore_signal(bsem, inc=1,
                                device_id=((my + d) % N_DEV,),
                                device_id_type=pl.DeviceIdType.MESH)
        pl.semaphore_wait(bsem, N_DEV - 1)

        wq_cp.wait()
        q_v[...] = (jnp.dot(x_ref[0].astype(BF), wq_v[...].astype(BF),
                            preferred_element_type=jnp.float32)
                    * SCALE).astype(BF)

        def attend(h):
            k_cp, v_cp = head_cps[h]
            k_cp.wait()
            v_cp.wait()
            kh = k_all[h].reshape(NKB // STRIDE, STRIDE, QB, DH)
            vh = v_all[h].reshape(NKB // STRIDE, STRIDE, QB, DH)
            for qb in range(NQB):
                ksel = kh[:, qb].reshape(KV_SEL, DH).astype(BF)
                vsel = vh[:, qb].reshape(KV_SEL, DH).astype(BF)
                qblk = q_v[qb * QB:(qb + 1) * QB, h * DH:(h + 1) * DH]
                s = jnp.dot(qblk, ksel.T, preferred_element_type=jnp.float32)
                e = jnp.exp(s)
                pv = jnp.dot(e.astype(BF), vsel,
                             preferred_element_type=jnp.float32)
                inv = 1.0 / jnp.sum(e, axis=1, keepdims=True)
                ctx_v[qb * QB:(qb + 1) * QB, h * DH:(h + 1) * DH] = (
                    pv * inv).astype(BF)

        def wave_send(part_ref, p1buf_ref, send_sems, recv_sems, cols):
            part_ref[...] = jnp.dot(
                ctx_v[:, cols], wo_v[cols, :].astype(BF),
                preferred_element_type=jnp.float32).astype(BF).reshape(
                    N_DEV, CHUNK, DM)
            rdmas = []
            for d in DORD:
                dst = (my + d) % N_DEV
                rd = pltpu.make_async_remote_copy(
                    src_ref=part_ref.at[pl.ds(dst, 1)],
                    dst_ref=p1buf_ref.at[pl.ds(my, 1)],
                    send_sem=send_sems.at[d - 1],
                    recv_sem=recv_sems.at[my],
                    device_id=(dst,),
                    device_id_type=pl.DeviceIdType.MESH,
                )
                rd.start()
                rdmas.append(rd)
            return rdmas

        def wave_collect(p1buf_ref, send_sems, recv_sems):
            for d in range(1, N_DEV):
                src = (my + d) % N_DEV
                pltpu.make_async_remote_copy(
                    src_ref=p1buf_ref.at[pl.ds(src, 1)],
                    dst_ref=p1buf_ref.at[pl.ds(src, 1)],
                    send_sem=send_sems.at[d - 1],
                    recv_sem=recv_sems.at[src],
                    device_id=(src,),
                    device_id_type=pl.DeviceIdType.MESH,
                ).wait_recv()
                red_v[...] += p1buf_ref[pl.ds(src, 1)][0].astype(jnp.float32)

        for h in range(H_LOC // 2):
            attend(h)
        wo_cp.wait()
        p1A_rdmas = wave_send(partA, p1bufA, p1A_send, p1A_recv,
                              slice(0, DM // 2))
        for h in range(H_LOC // 2, H_LOC):
            attend(h)
        p1B_rdmas = wave_send(partB, p1bufB, p1B_send, p1B_recv,
                              slice(DM // 2, DM))

        red_v[...] = (partA[pl.ds(my, 1)][0].astype(jnp.float32)
                      + partB[pl.ds(my, 1)][0].astype(jnp.float32))
        wave_collect(p1bufA, p1A_send, p1A_recv)
        wave_collect(p1bufB, p1B_send, p1B_recv)

        p2buf[pl.ds(my, 1)] = red_v[...].astype(BF)[None]
        p2_rdmas = []
        for d in DORD:
            dst = (my + d) % N_DEV
            rd = pltpu.make_async_remote_copy(
                src_ref=p2buf.at[pl.ds(my, 1)],
                dst_ref=p2buf.at[pl.ds(my, 1)],
                send_sem=p2_send.at[d - 1],
                recv_sem=p2_recv.at[my],
                device_id=(dst,),
                device_id_type=pl.DeviceIdType.MESH,
            )
            rd.start()
            p2_rdmas.append(rd)
        for d in range(1, N_DEV):
            src = (my + d) % N_DEV
            pltpu.make_async_remote_copy(
                src_ref=p2buf.at[pl.ds(src, 1)],
                dst_ref=p2buf.at[pl.ds(src, 1)],
                send_sem=p2_send.at[d - 1],
                recv_sem=p2_recv.at[src],
                device_id=(src,),
                device_id_type=pl.DeviceIdType.MESH,
            ).wait_recv()
        out_ref[0] = p2buf[...].reshape(SQ, DM).astype(jnp.float32)
        for rd in p1A_rdmas + p1B_rdmas + p2_rdmas:
            rd.wait_send()

    return pl.pallas_call(
        body,
        out_shape=jax.ShapeDtypeStruct((1, SQ, DM), jnp.float32),
        in_specs=[
            pl.BlockSpec(memory_space=pltpu.VMEM),
            pl.BlockSpec(memory_space=pl.ANY),
            pl.BlockSpec(memory_space=pl.ANY),
            pl.BlockSpec(memory_space=pl.ANY),
            pl.BlockSpec(memory_space=pl.ANY),
        ],
        out_specs=pl.BlockSpec(memory_space=pltpu.VMEM),
        scratch_shapes=[
            pltpu.VMEM((DM, DM), jnp.float32),
            pltpu.VMEM((DM, DM), jnp.float32),
            pltpu.VMEM((H_LOC, SKV, DH), jnp.float32),
            pltpu.VMEM((H_LOC, SKV, DH), jnp.float32),
            pltpu.VMEM((SQ, DM), BF),
            pltpu.VMEM((SQ, DM), BF),
            pltpu.VMEM((N_DEV, CHUNK, DM), BF),
            pltpu.VMEM((N_DEV, CHUNK, DM), BF),
            pltpu.VMEM((CHUNK, DM), jnp.float32),
            pltpu.VMEM((N_DEV, CHUNK, DM), BF),
            pltpu.VMEM((N_DEV, CHUNK, DM), BF),
            pltpu.VMEM((N_DEV, CHUNK, DM), BF),
            pltpu.SemaphoreType.DMA((2,)),
            pltpu.SemaphoreType.DMA((H_LOC,)),
            pltpu.SemaphoreType.DMA((H_LOC,)),
            pltpu.SemaphoreType.DMA((N_DEV - 1,)),
            pltpu.SemaphoreType.DMA((N_DEV,)),
            pltpu.SemaphoreType.DMA((N_DEV - 1,)),
            pltpu.SemaphoreType.DMA((N_DEV,)),
            pltpu.SemaphoreType.DMA((N_DEV - 1,)),
            pltpu.SemaphoreType.DMA((N_DEV,)),
        ],
        compiler_params=pltpu.CompilerParams(
            collective_id=0, vmem_limit_bytes=100 * 1024 * 1024),
    )(x, Wq, K_ext, V_ext, Wo)
